# Initial kernel scaffold; baseline (speedup 1.0000x reference)
import jax
import jax.numpy as jnp
from jax.experimental import pallas as pl


def kernel(mels, conv_w, ln_w0, ln_b0, ln_w1, ln_b1, ln_w2, ln_b2, ln_w3, ln_b3, ln_w4, ln_b4, lin_w0, lin_w1, lin_w2, lin_w3, lin_w4, lin_b4, embedding, W_ih, W_hh, b_ih, b_hh):
    raise NotImplementedError("write your pallas kernel here")


# speedup vs baseline: 2.3275x; 2.3275x over previous
import functools

import jax
import jax.numpy as jnp
from jax import lax
from jax.experimental import pallas as pl
from jax.experimental.pallas import tpu as pltpu

_B, _IN_CH, _T = 32, 80, 1024
_C, _M, _ZD, _CD = 768, 512, 64, 256
_TO = _T // 2
_N = _B * _TO
_TB = 1024
_NB = _N // _TB
_TC = 128
_G4 = 4 * _CD

_dot = functools.partial(jnp.dot, preferred_element_type=jnp.float32)


def _ln_ref(x, w, b):
    mu = jnp.mean(x, axis=-1, keepdims=True)
    var = jnp.var(x, axis=-1, keepdims=True)
    return (x - mu) / jnp.sqrt(var + 1e-5) * w + b


def _vq_kernel(zf_ref, emb_ref, embt_ref, e2_ref, wiht_ref, bihh_ref,
               zst_ref, xproj_ref, lossp_ref):
    zf = zf_ref[...]
    x2 = jnp.sum(zf * zf, axis=1, keepdims=True)
    scores = _dot(zf, embt_ref[...])
    dist = (e2_ref[...] + x2) - 2.0 * scores
    idx = jnp.argmin(dist, axis=1)
    oh = (idx[:, None] == lax.broadcasted_iota(jnp.int32, (_TB, _M), 1))
    q = _dot(oh.astype(jnp.float32), emb_ref[...])
    zst = zf + (q - zf)
    zst_ref[...] = zst
    lossp_ref[...] = jnp.full((1, 8, 128), jnp.sum((zf - q) ** 2),
                              dtype=jnp.float32)
    xproj_ref[...] = _dot(zst, wiht_ref[...]) + bihh_ref[...]


def _lstm_kernel(xp_ref, whht_ref, out_ref, h_ref, c_ref):
    @pl.when(pl.program_id(0) == 0)
    def _init():
        h_ref[...] = jnp.zeros((_B, _CD), jnp.float32)
        c_ref[...] = jnp.zeros((_B, _CD), jnp.float32)

    def body(t, _):
        g = xp_ref[:, t, :] + _dot(h_ref[...], whht_ref[...])
        i_g = g[:, :_CD]
        f_g = g[:, _CD:2 * _CD]
        g_g = g[:, 2 * _CD:3 * _CD]
        o_g = g[:, 3 * _CD:]
        c = jax.nn.sigmoid(f_g) * c_ref[...] + jax.nn.sigmoid(i_g) * jnp.tanh(g_g)
        h = jax.nn.sigmoid(o_g) * jnp.tanh(c)
        c_ref[...] = c
        h_ref[...] = h
        out_ref[:, t, :] = h
        return 0

    lax.fori_loop(0, _TC, body, 0, unroll=False)


def _full(i):
    return pl.BlockSpec(i, lambda g: tuple(0 for _ in i))


def kernel(mels, conv_w, ln_w0, ln_b0, ln_w1, ln_b1, ln_w2, ln_b2, ln_w3, ln_b3, ln_w4, ln_b4, lin_w0, lin_w1, lin_w2, lin_w3, lin_w4, lin_b4, embedding, W_ih, W_hh, b_ih, b_hh):
    f32 = jnp.float32
    z = lax.conv_general_dilated(mels, conv_w, window_strides=(2,), padding=[(1, 1)], dimension_numbers=("NCH", "OIH", "NCH"))
    z = jnp.swapaxes(z, 1, 2)
    lnw_l = [ln_w0, ln_w1, ln_w2, ln_w3, ln_w4]
    lnb_l = [ln_b0, ln_b1, ln_b2, ln_b3, ln_b4]
    lw = [lin_w0, lin_w1, lin_w2, lin_w3]
    for i in range(4):
        z = _ln_ref(z, lnw_l[i], lnb_l[i])
        z = jax.nn.relu(z)
        z = z @ lw[i].T
    z = _ln_ref(z, lnw_l[4], lnb_l[4])
    z = jax.nn.relu(z)
    z = z @ lin_w4.T + lin_b4

    x_flat = lax.stop_gradient(z).reshape(-1, _ZD)
    distances = jnp.sum(embedding ** 2, axis=1)[None, :] + jnp.sum(x_flat ** 2, axis=1, keepdims=True) - 2.0 * (x_flat @ embedding.T)
    indices = jnp.argmin(distances, axis=-1)
    quantized = jnp.take(embedding, indices, axis=0).reshape(z.shape)
    e_latent_loss = jnp.mean((z - lax.stop_gradient(quantized)) ** 2)
    loss = 0.25 * e_latent_loss
    z_st = z + lax.stop_gradient(quantized - z)
    xproj = (z_st.reshape(_N, _ZD) @ W_ih.T + (b_ih + b_hh)[None, :])

    c_seq = pl.pallas_call(
        _lstm_kernel,
        grid=(_TO // _TC,),
        in_specs=[
            pl.BlockSpec((_B, _TC, _G4), lambda i: (0, i, 0)),
            _full((_CD, _G4)),
        ],
        out_specs=pl.BlockSpec((_B, _TC, _CD), lambda i: (0, i, 0)),
        out_shape=jax.ShapeDtypeStruct((_B, _TO, _CD), f32),
        scratch_shapes=[
            pltpu.VMEM((_B, _CD), f32),
            pltpu.VMEM((_B, _CD), f32),
        ],
        compiler_params=pltpu.CompilerParams(
            dimension_semantics=("arbitrary",)),
    )(xproj.reshape(_B, _TO, _G4), W_hh.T)

    return (z_st, c_seq, loss)


# speedup vs baseline: 2.5340x; 1.0887x over previous
import functools

import jax
import jax.numpy as jnp
from jax import lax
from jax.experimental import pallas as pl
from jax.experimental.pallas import tpu as pltpu

_B, _IN_CH, _T = 32, 80, 1024
_C, _M, _ZD, _CD = 768, 512, 64, 256
_TO = _T // 2
_N = _B * _TO
_TB = 1024
_NB = _N // _TB
_TC = 128
_G4 = 4 * _CD

_dot = functools.partial(jnp.dot, preferred_element_type=jnp.float32)


def _ln_ref(x, w, b):
    mu = jnp.mean(x, axis=-1, keepdims=True)
    var = jnp.var(x, axis=-1, keepdims=True)
    return (x - mu) / jnp.sqrt(var + 1e-5) * w + b


def _vq_kernel(zf_ref, emb_ref, embt_ref, e2_ref, wiht_ref, bihh_ref,
               zst_ref, xproj_ref, lossp_ref):
    zf = zf_ref[...]
    x2 = jnp.sum(zf * zf, axis=1, keepdims=True)
    scores = _dot(zf, embt_ref[...])
    dist = (e2_ref[...] + x2) - 2.0 * scores
    mval = jnp.min(dist, axis=1, keepdims=True)
    miota = lax.broadcasted_iota(jnp.int32, (_TB, _M), 1)
    idx = jnp.min(jnp.where(dist == mval, miota, _M), axis=1)
    oh = (idx[:, None] == miota)
    q = _dot(oh.astype(jnp.float32), emb_ref[...])
    zst = zf + (q - zf)
    zst_ref[...] = zst
    lossp_ref[...] = jnp.full((1, 8, 128), jnp.sum((zf - q) ** 2),
                              dtype=jnp.float32)
    xproj_ref[...] = _dot(zst, wiht_ref[...]) + bihh_ref[...]


def _lstm_kernel(xp_ref, whht_ref, out_ref, h_ref, c_ref):
    @pl.when(pl.program_id(0) == 0)
    def _init():
        h_ref[...] = jnp.zeros((_B, _CD), jnp.float32)
        c_ref[...] = jnp.zeros((_B, _CD), jnp.float32)

    def body(t, _):
        g = xp_ref[:, t, :] + _dot(h_ref[...], whht_ref[...])
        i_g = g[:, :_CD]
        f_g = g[:, _CD:2 * _CD]
        g_g = g[:, 2 * _CD:3 * _CD]
        o_g = g[:, 3 * _CD:]
        c = jax.nn.sigmoid(f_g) * c_ref[...] + jax.nn.sigmoid(i_g) * jnp.tanh(g_g)
        h = jax.nn.sigmoid(o_g) * jnp.tanh(c)
        c_ref[...] = c
        h_ref[...] = h
        out_ref[:, t, :] = h
        return 0

    lax.fori_loop(0, _TC, body, 0, unroll=False)


def _full(i):
    return pl.BlockSpec(i, lambda g: tuple(0 for _ in i))


def kernel(mels, conv_w, ln_w0, ln_b0, ln_w1, ln_b1, ln_w2, ln_b2, ln_w3, ln_b3, ln_w4, ln_b4, lin_w0, lin_w1, lin_w2, lin_w3, lin_w4, lin_b4, embedding, W_ih, W_hh, b_ih, b_hh):
    f32 = jnp.float32
    z = lax.conv_general_dilated(mels, conv_w, window_strides=(2,), padding=[(1, 1)], dimension_numbers=("NCH", "OIH", "NCH"))
    z = jnp.swapaxes(z, 1, 2)
    lnw_l = [ln_w0, ln_w1, ln_w2, ln_w3, ln_w4]
    lnb_l = [ln_b0, ln_b1, ln_b2, ln_b3, ln_b4]
    lw = [lin_w0, lin_w1, lin_w2, lin_w3]
    for i in range(4):
        z = _ln_ref(z, lnw_l[i], lnb_l[i])
        z = jax.nn.relu(z)
        z = z @ lw[i].T
    z = _ln_ref(z, lnw_l[4], lnb_l[4])
    z = jax.nn.relu(z)
    z = z @ lin_w4.T + lin_b4

    x_flat = z.reshape(_N, _ZD)
    e2 = jnp.sum(embedding ** 2, axis=1)[None, :]
    zst, xproj, lossp = pl.pallas_call(
        _vq_kernel,
        grid=(_NB,),
        in_specs=[
            pl.BlockSpec((_TB, _ZD), lambda i: (i, 0)),
            _full((_M, _ZD)),
            _full((_ZD, _M)),
            _full((1, _M)),
            _full((_ZD, _G4)),
            _full((1, _G4)),
        ],
        out_specs=[
            pl.BlockSpec((_TB, _ZD), lambda i: (i, 0)),
            pl.BlockSpec((_TB, _G4), lambda i: (i, 0)),
            pl.BlockSpec((1, 8, 128), lambda i: (i, 0, 0)),
        ],
        out_shape=[
            jax.ShapeDtypeStruct((_N, _ZD), f32),
            jax.ShapeDtypeStruct((_N, _G4), f32),
            jax.ShapeDtypeStruct((_NB, 8, 128), f32),
        ],
        compiler_params=pltpu.CompilerParams(
            dimension_semantics=("parallel",)),
    )(x_flat, embedding, embedding.T, e2, W_ih.T, (b_ih + b_hh)[None, :])

    loss = 0.25 * jnp.sum(lossp[:, 0, 0]) / (_N * _ZD)
    z_st = zst.reshape(_B, _TO, _ZD)

    c_seq = pl.pallas_call(
        _lstm_kernel,
        grid=(_TO // _TC,),
        in_specs=[
            pl.BlockSpec((_B, _TC, _G4), lambda i: (0, i, 0)),
            _full((_CD, _G4)),
        ],
        out_specs=pl.BlockSpec((_B, _TC, _CD), lambda i: (0, i, 0)),
        out_shape=jax.ShapeDtypeStruct((_B, _TO, _CD), f32),
        scratch_shapes=[
            pltpu.VMEM((_B, _CD), f32),
            pltpu.VMEM((_B, _CD), f32),
        ],
        compiler_params=pltpu.CompilerParams(
            dimension_semantics=("arbitrary",)),
    )(xproj.reshape(_B, _TO, _G4), W_hh.T)

    return (z_st, c_seq, loss)


# speedup vs baseline: 2.5976x; 1.0251x over previous
import functools

import jax
import jax.numpy as jnp
from jax import lax
from jax.experimental import pallas as pl
from jax.experimental.pallas import tpu as pltpu

_B, _IN_CH, _T = 32, 80, 1024
_C, _M, _ZD, _CD = 768, 512, 64, 256
_TO = _T // 2
_N = _B * _TO
_TC = 128
_NT = _B * _TC
_G4 = 4 * _CD

_dot = functools.partial(jnp.dot, preferred_element_type=jnp.float32)


def _ln(x, w, b):
    mu = jnp.mean(x, axis=-1, keepdims=True)
    var = jnp.var(x, axis=-1, keepdims=True)
    return (x - mu) / jnp.sqrt(var + 1e-5) * w + b


def _vq_lstm_kernel(z_ref, emb_ref, embt_ref, e2_ref, wiht_ref, bihh_ref,
                    whht_ref, zst_ref, cseq_ref, lossp_ref,
                    xp_ref, h_ref, c_ref):
    zf = z_ref[...].reshape(_NT, _ZD)
    x2 = jnp.sum(zf * zf, axis=1, keepdims=True)
    scores = _dot(zf, embt_ref[...])
    dist = (e2_ref[...] + x2) - 2.0 * scores
    mval = jnp.min(dist, axis=1, keepdims=True)
    miota = lax.broadcasted_iota(jnp.int32, (_NT, _M), 1)
    idx = jnp.min(jnp.where(dist == mval, miota, _M), axis=1)
    oh = (idx[:, None] == miota)
    q = _dot(oh.astype(jnp.float32), emb_ref[...])
    zst = zf + (q - zf)
    zst_ref[...] = zst.reshape(_B, _TC, _ZD)
    lossp_ref[...] = jnp.full((1, 8, 128), jnp.sum((zf - q) ** 2),
                              dtype=jnp.float32)
    xp_ref[...] = (_dot(zst, wiht_ref[...]) + bihh_ref[...]).reshape(
        _B, _TC, _G4)

    @pl.when(pl.program_id(0) == 0)
    def _init():
        h_ref[...] = jnp.zeros((_B, _CD), jnp.float32)
        c_ref[...] = jnp.zeros((_B, _CD), jnp.float32)

    def body(t, _):
        g = xp_ref[:, t, :] + _dot(h_ref[...], whht_ref[...])
        i_g = g[:, :_CD]
        f_g = g[:, _CD:2 * _CD]
        g_g = g[:, 2 * _CD:3 * _CD]
        o_g = g[:, 3 * _CD:]
        c = jax.nn.sigmoid(f_g) * c_ref[...] + jax.nn.sigmoid(i_g) * jnp.tanh(g_g)
        h = jax.nn.sigmoid(o_g) * jnp.tanh(c)
        c_ref[...] = c
        h_ref[...] = h
        cseq_ref[:, t, :] = h
        return 0

    lax.fori_loop(0, _TC, body, 0, unroll=False)


def _full(i):
    return pl.BlockSpec(i, lambda g: tuple(0 for _ in i))


def kernel(mels, conv_w, ln_w0, ln_b0, ln_w1, ln_b1, ln_w2, ln_b2, ln_w3, ln_b3, ln_w4, ln_b4, lin_w0, lin_w1, lin_w2, lin_w3, lin_w4, lin_b4, embedding, W_ih, W_hh, b_ih, b_hh):
    f32 = jnp.float32
    z = lax.conv_general_dilated(mels, conv_w, window_strides=(2,), padding=[(1, 1)], dimension_numbers=("NCH", "OIH", "NCH"))
    z = jnp.swapaxes(z, 1, 2)
    lnw_l = [ln_w0, ln_w1, ln_w2, ln_w3, ln_w4]
    lnb_l = [ln_b0, ln_b1, ln_b2, ln_b3, ln_b4]
    lw = [lin_w0, lin_w1, lin_w2, lin_w3]
    for i in range(4):
        z = _ln(z, lnw_l[i], lnb_l[i])
        z = jax.nn.relu(z)
        z = z @ lw[i].T
    z = _ln(z, lnw_l[4], lnb_l[4])
    z = jax.nn.relu(z)
    z = z @ lin_w4.T + lin_b4

    e2 = jnp.sum(embedding ** 2, axis=1)[None, :]
    z_st, c_seq, lossp = pl.pallas_call(
        _vq_lstm_kernel,
        grid=(_TO // _TC,),
        in_specs=[
            pl.BlockSpec((_B, _TC, _ZD), lambda i: (0, i, 0)),
            _full((_M, _ZD)),
            _full((_ZD, _M)),
            _full((1, _M)),
            _full((_ZD, _G4)),
            _full((1, _G4)),
            _full((_CD, _G4)),
        ],
        out_specs=[
            pl.BlockSpec((_B, _TC, _ZD), lambda i: (0, i, 0)),
            pl.BlockSpec((_B, _TC, _CD), lambda i: (0, i, 0)),
            pl.BlockSpec((1, 8, 128), lambda i: (i, 0, 0)),
        ],
        out_shape=[
            jax.ShapeDtypeStruct((_B, _TO, _ZD), f32),
            jax.ShapeDtypeStruct((_B, _TO, _CD), f32),
            jax.ShapeDtypeStruct((_TO // _TC, 8, 128), f32),
        ],
        scratch_shapes=[
            pltpu.VMEM((_B, _TC, _G4), f32),
            pltpu.VMEM((_B, _CD), f32),
            pltpu.VMEM((_B, _CD), f32),
        ],
        compiler_params=pltpu.CompilerParams(
            dimension_semantics=("arbitrary",)),
    )(z, embedding, embedding.T, e2, W_ih.T, (b_ih + b_hh)[None, :], W_hh.T)

    loss = 0.25 * jnp.sum(lossp[:, 0, 0]) / (_N * _ZD)
    return (z_st, c_seq, loss)


# speedup vs baseline: 2.6719x; 1.0286x over previous
import functools

import jax
import jax.numpy as jnp
from jax import lax
from jax.experimental import pallas as pl
from jax.experimental.pallas import tpu as pltpu

_B, _IN_CH, _T = 32, 80, 1024
_C, _M, _ZD, _CD = 768, 512, 64, 256
_TO = _T // 2
_N = _B * _TO
_TC = 128
_NT = _B * _TC
_G4 = 4 * _CD

_dot = functools.partial(jnp.dot, preferred_element_type=jnp.float32)


def _ln(x, w, b):
    mu = jnp.mean(x, axis=-1, keepdims=True)
    var = jnp.var(x, axis=-1, keepdims=True)
    return (x - mu) / jnp.sqrt(var + 1e-5) * w + b


def _vq_lstm_kernel(z_ref, emb_ref, embt_ref, e2_ref, wiht_ref, bihh_ref,
                    whht_ref, zst_ref, cseq_ref, lossp_ref,
                    xp_ref, h_ref, c_ref):
    zf = z_ref[...].reshape(_NT, _ZD)
    x2 = jnp.sum(zf * zf, axis=1, keepdims=True)
    scores = _dot(zf, embt_ref[...])
    dist = (e2_ref[...] + x2) - 2.0 * scores
    mval = jnp.min(dist, axis=1, keepdims=True)
    miota = lax.broadcasted_iota(jnp.int32, (_NT, _M), 1)
    idx = jnp.min(jnp.where(dist == mval, miota, _M), axis=1)
    oh = (idx[:, None] == miota)
    q = _dot(oh.astype(jnp.float32), emb_ref[...])
    zst = zf + (q - zf)
    zst_ref[...] = zst.reshape(_B, _TC, _ZD)
    lossp_ref[...] = jnp.full((1, 8, 128), jnp.sum((zf - q) ** 2),
                              dtype=jnp.float32)
    xp_ref[...] = (_dot(zst, wiht_ref[...]) + bihh_ref[...]).reshape(
        _B, _TC, _G4)

    @pl.when(pl.program_id(0) == 0)
    def _init():
        h_ref[...] = jnp.zeros((_B, _CD), jnp.float32)
        c_ref[...] = jnp.zeros((_B, _CD), jnp.float32)

    def body(t, _):
        g = xp_ref[:, t, :] + _dot(h_ref[...], whht_ref[...])
        i_g = g[:, :_CD]
        f_g = g[:, _CD:2 * _CD]
        g_g = g[:, 2 * _CD:3 * _CD]
        o_g = g[:, 3 * _CD:]
        c = jax.nn.sigmoid(f_g) * c_ref[...] + jax.nn.sigmoid(i_g) * jnp.tanh(g_g)
        h = jax.nn.sigmoid(o_g) * jnp.tanh(c)
        c_ref[...] = c
        h_ref[...] = h
        cseq_ref[:, t, :] = h
        return 0

    lax.fori_loop(0, _TC, body, 0, unroll=4)


def _full(i):
    return pl.BlockSpec(i, lambda g: tuple(0 for _ in i))


def kernel(mels, conv_w, ln_w0, ln_b0, ln_w1, ln_b1, ln_w2, ln_b2, ln_w3, ln_b3, ln_w4, ln_b4, lin_w0, lin_w1, lin_w2, lin_w3, lin_w4, lin_b4, embedding, W_ih, W_hh, b_ih, b_hh):
    f32 = jnp.float32
    z = lax.conv_general_dilated(mels, conv_w, window_strides=(2,), padding=[(1, 1)], dimension_numbers=("NCH", "OIH", "NCH"))
    z = jnp.swapaxes(z, 1, 2)
    lnw_l = [ln_w0, ln_w1, ln_w2, ln_w3, ln_w4]
    lnb_l = [ln_b0, ln_b1, ln_b2, ln_b3, ln_b4]
    lw = [lin_w0, lin_w1, lin_w2, lin_w3]
    for i in range(4):
        z = _ln(z, lnw_l[i], lnb_l[i])
        z = jax.nn.relu(z)
        z = z @ lw[i].T
    z = _ln(z, lnw_l[4], lnb_l[4])
    z = jax.nn.relu(z)
    z = z @ lin_w4.T + lin_b4

    e2 = jnp.sum(embedding ** 2, axis=1)[None, :]
    z_st, c_seq, lossp = pl.pallas_call(
        _vq_lstm_kernel,
        grid=(_TO // _TC,),
        in_specs=[
            pl.BlockSpec((_B, _TC, _ZD), lambda i: (0, i, 0)),
            _full((_M, _ZD)),
            _full((_ZD, _M)),
            _full((1, _M)),
            _full((_ZD, _G4)),
            _full((1, _G4)),
            _full((_CD, _G4)),
        ],
        out_specs=[
            pl.BlockSpec((_B, _TC, _ZD), lambda i: (0, i, 0)),
            pl.BlockSpec((_B, _TC, _CD), lambda i: (0, i, 0)),
            pl.BlockSpec((1, 8, 128), lambda i: (i, 0, 0)),
        ],
        out_shape=[
            jax.ShapeDtypeStruct((_B, _TO, _ZD), f32),
            jax.ShapeDtypeStruct((_B, _TO, _CD), f32),
            jax.ShapeDtypeStruct((_TO // _TC, 8, 128), f32),
        ],
        scratch_shapes=[
            pltpu.VMEM((_B, _TC, _G4), f32),
            pltpu.VMEM((_B, _CD), f32),
            pltpu.VMEM((_B, _CD), f32),
        ],
        compiler_params=pltpu.CompilerParams(
            dimension_semantics=("arbitrary",)),
    )(z, embedding, embedding.T, e2, W_ih.T, (b_ih + b_hh)[None, :], W_hh.T)

    loss = 0.25 * jnp.sum(lossp[:, 0, 0]) / (_N * _ZD)
    return (z_st, c_seq, loss)


# speedup vs baseline: 2.6770x; 1.0019x over previous
import functools

import jax
import jax.numpy as jnp
from jax import lax
from jax.experimental import pallas as pl
from jax.experimental.pallas import tpu as pltpu

_B, _IN_CH, _T = 32, 80, 1024
_C, _M, _ZD, _CD = 768, 512, 64, 256
_TO = _T // 2
_N = _B * _TO
_TC = 64
_NT = _B * _TC
_G4 = 4 * _CD

_dot = functools.partial(jnp.dot, preferred_element_type=jnp.float32)


def _ln(x, w, b):
    mu = jnp.mean(x, axis=-1, keepdims=True)
    var = jnp.var(x, axis=-1, keepdims=True)
    return (x - mu) / jnp.sqrt(var + 1e-5) * w + b


def _vq_lstm_kernel(z_ref, emb_ref, embt_ref, e2_ref, wiht_ref, bihh_ref,
                    whht_ref, zst_ref, cseq_ref, lossp_ref,
                    xp_ref, h_ref, c_ref):
    zf = z_ref[...].reshape(_NT, _ZD)
    x2 = jnp.sum(zf * zf, axis=1, keepdims=True)
    scores = _dot(zf, embt_ref[...])
    dist = (e2_ref[...] + x2) - 2.0 * scores
    mval = jnp.min(dist, axis=1, keepdims=True)
    miota = lax.broadcasted_iota(jnp.int32, (_NT, _M), 1)
    idx = jnp.min(jnp.where(dist == mval, miota, _M), axis=1)
    oh = (idx[:, None] == miota)
    q = _dot(oh.astype(jnp.float32), emb_ref[...])
    zst = zf + (q - zf)
    zst_ref[...] = zst.reshape(_B, _TC, _ZD)
    lossp_ref[...] = jnp.full((1, 8, 128), jnp.sum((zf - q) ** 2),
                              dtype=jnp.float32)
    xp_ref[...] = (_dot(zst, wiht_ref[...]) + bihh_ref[...]).reshape(
        _B, _TC, _G4)

    @pl.when(pl.program_id(0) == 0)
    def _init():
        h_ref[...] = jnp.zeros((_B, _CD), jnp.float32)
        c_ref[...] = jnp.zeros((_B, _CD), jnp.float32)

    def body(t, _):
        g = xp_ref[:, t, :] + _dot(h_ref[...], whht_ref[...])
        i_g = g[:, :_CD]
        f_g = g[:, _CD:2 * _CD]
        g_g = g[:, 2 * _CD:3 * _CD]
        o_g = g[:, 3 * _CD:]
        c = jax.nn.sigmoid(f_g) * c_ref[...] + jax.nn.sigmoid(i_g) * jnp.tanh(g_g)
        h = jax.nn.sigmoid(o_g) * jnp.tanh(c)
        c_ref[...] = c
        h_ref[...] = h
        cseq_ref[:, t, :] = h
        return 0

    lax.fori_loop(0, _TC, body, 0, unroll=8)


def _full(i):
    return pl.BlockSpec(i, lambda g: tuple(0 for _ in i))


def kernel(mels, conv_w, ln_w0, ln_b0, ln_w1, ln_b1, ln_w2, ln_b2, ln_w3, ln_b3, ln_w4, ln_b4, lin_w0, lin_w1, lin_w2, lin_w3, lin_w4, lin_b4, embedding, W_ih, W_hh, b_ih, b_hh):
    f32 = jnp.float32
    z = lax.conv_general_dilated(mels, conv_w, window_strides=(2,), padding=[(1, 1)], dimension_numbers=("NCH", "OIH", "NCH"))
    z = jnp.swapaxes(z, 1, 2)
    lnw_l = [ln_w0, ln_w1, ln_w2, ln_w3, ln_w4]
    lnb_l = [ln_b0, ln_b1, ln_b2, ln_b3, ln_b4]
    lw = [lin_w0, lin_w1, lin_w2, lin_w3]
    for i in range(4):
        z = _ln(z, lnw_l[i], lnb_l[i])
        z = jax.nn.relu(z)
        z = z @ lw[i].T
    z = _ln(z, lnw_l[4], lnb_l[4])
    z = jax.nn.relu(z)
    z = z @ lin_w4.T + lin_b4

    e2 = jnp.sum(embedding ** 2, axis=1)[None, :]
    z_st, c_seq, lossp = pl.pallas_call(
        _vq_lstm_kernel,
        grid=(_TO // _TC,),
        in_specs=[
            pl.BlockSpec((_B, _TC, _ZD), lambda i: (0, i, 0)),
            _full((_M, _ZD)),
            _full((_ZD, _M)),
            _full((1, _M)),
            _full((_ZD, _G4)),
            _full((1, _G4)),
            _full((_CD, _G4)),
        ],
        out_specs=[
            pl.BlockSpec((_B, _TC, _ZD), lambda i: (0, i, 0)),
            pl.BlockSpec((_B, _TC, _CD), lambda i: (0, i, 0)),
            pl.BlockSpec((1, 8, 128), lambda i: (i, 0, 0)),
        ],
        out_shape=[
            jax.ShapeDtypeStruct((_B, _TO, _ZD), f32),
            jax.ShapeDtypeStruct((_B, _TO, _CD), f32),
            jax.ShapeDtypeStruct((_TO // _TC, 8, 128), f32),
        ],
        scratch_shapes=[
            pltpu.VMEM((_B, _TC, _G4), f32),
            pltpu.VMEM((_B, _CD), f32),
            pltpu.VMEM((_B, _CD), f32),
        ],
        compiler_params=pltpu.CompilerParams(
            dimension_semantics=("arbitrary",)),
    )(z, embedding, embedding.T, e2, W_ih.T, (b_ih + b_hh)[None, :], W_hh.T)

    loss = 0.25 * jnp.sum(lossp[:, 0, 0]) / (_N * _ZD)
    return (z_st, c_seq, loss)
